# baseline (speedup 1.0000x reference)
import functools

import jax
import jax.numpy as jnp
from jax import lax
from jax.experimental import pallas as pl
from jax.experimental.pallas import tpu as pltpu
from jax.experimental.pallas import tpu_sc as plsc

F32 = jnp.float32
HI = lax.Precision.HIGHEST

N_NODE = 4096
FEAT = 128
HID = 32
NREL = 5
NTAB = 2 * NREL
NEDGE = 65536
SHID_DIM = 64
OUT_DIM = 64

NCORES = 2
NSUB = 16
NTILES = NCORES * NSUB
EPT = NEDGE // NTILES
CHUNK = 128
NCHUNK = EPT // CHUNK
ROWS_PT = N_NODE // NSUB

BV = 512


def _tables_body(fv_ref, fu_ref, w_ref, o_ref):
    d = pl.program_id(0)

    @pl.when(d < NREL)
    def _():
        o_ref[0] = jnp.dot(fv_ref[...], w_ref[0], preferred_element_type=F32,
                           precision=lax.Precision.DEFAULT)

    @pl.when(d >= NREL)
    def _():
        o_ref[0] = jnp.dot(fu_ref[...], w_ref[0], preferred_element_type=F32,
                           precision=lax.Precision.DEFAULT)


def _make_tables(fv, fu, W):
    return pl.pallas_call(
        _tables_body,
        grid=(NTAB,),
        in_specs=[
            pl.BlockSpec((N_NODE, FEAT), lambda d: (0, 0)),
            pl.BlockSpec((N_NODE, FEAT), lambda d: (0, 0)),
            pl.BlockSpec((1, FEAT, HID), lambda d: (d % NREL, 0, 0)),
        ],
        out_specs=pl.BlockSpec((1, N_NODE, HID), lambda d: (d, 0, 0)),
        out_shape=jax.ShapeDtypeStruct((NTAB, N_NODE, HID), F32),
    )(fv, fu, W)


def _seg_body(tbl_ref, ucol_ref, urow_ref, uval_ref, vcol_ref, vrow_ref,
              vval_ref, out_ref,
              acc_ref, colb, rowb, valb, rbuf0, rbuf1, zbuf, sem0, sem1,
              ssem0, ssem1):
    c = lax.axis_index("c")
    s = lax.axis_index("s")
    wid = c * NSUB + s

    def zero_row(r, carry):
        zv = jnp.zeros((16,), F32)
        zbuf[r, pl.ds(0, 16)] = zv
        zbuf[r, pl.ds(16, 16)] = zv
        return carry
    lax.fori_loop(0, ROWS_PT, zero_row, 0)

    def zero_acc(d, carry):
        pltpu.sync_copy(zbuf, acc_ref.at[d, pl.ds(s * ROWS_PT, ROWS_PT)])
        return carry
    lax.fori_loop(0, NTAB, zero_acc, 0)
    plsc.subcore_barrier()

    def scale(rbuf, k):
        for g in range(CHUNK // 16):
            v16 = valb[pl.ds(k * CHUNK + g * 16, 16)]
            for j in range(16):
                e = g * 16 + j
                sc = v16[j]
                rbuf[e, pl.ds(0, 16)] = rbuf[e, pl.ds(0, 16)] * sc
                rbuf[e, pl.ds(16, 16)] = rbuf[e, pl.ds(16, 16)] * sc

    def gather(d, k, rbuf, sem):
        pltpu.async_copy(tbl_ref.at[d].at[colb.at[pl.ds(k * CHUNK, CHUNK)]],
                         rbuf, sem)

    def gather_wait(d, k, rbuf, sem):
        pltpu.make_async_copy(
            tbl_ref.at[d].at[colb.at[pl.ds(k * CHUNK, CHUNK)]], rbuf,
            sem).wait()

    def scatter_wait(d, k, rbuf, ssem):
        pltpu.make_async_copy(rbuf, acc_ref.at[d].at[rowb.at[k]],
                              ssem).wait()

    def make_phase(col_ref, row_ref, val_ref, off):
        def phase(i, carry):
            d = i + off
            pltpu.sync_copy(col_ref.at[i, pl.ds(wid * EPT, EPT)], colb)
            pltpu.sync_copy(row_ref.at[i, pl.ds(wid * NCHUNK, NCHUNK)], rowb)
            pltpu.sync_copy(val_ref.at[i, pl.ds(wid * EPT, EPT)], valb)
            gather(d, 0, rbuf0, sem0)
            gather(d, 1, rbuf1, sem1)

            def pair(p, carry2):
                k0 = 2 * p
                gather_wait(d, k0, rbuf0, sem0)
                scale(rbuf0, k0)
                pltpu.async_copy(rbuf0, acc_ref.at[d].at[rowb.at[k0]], ssem0,
                                 add=True)
                gather_wait(d, k0 + 1, rbuf1, sem1)
                scale(rbuf1, k0 + 1)
                pltpu.async_copy(rbuf1, acc_ref.at[d].at[rowb.at[k0 + 1]],
                                 ssem1, add=True)

                @pl.when(p < NCHUNK // 2 - 1)
                def _():
                    scatter_wait(d, k0, rbuf0, ssem0)
                    gather(d, k0 + 2, rbuf0, sem0)
                    scatter_wait(d, k0 + 1, rbuf1, ssem1)
                    gather(d, k0 + 3, rbuf1, sem1)
                return carry2
            lax.fori_loop(0, NCHUNK // 2, pair, 0)
            scatter_wait(d, NCHUNK - 2, rbuf0, ssem0)
            scatter_wait(d, NCHUNK - 1, rbuf1, ssem1)
            return carry
        return phase

    lax.fori_loop(0, NREL, make_phase(ucol_ref, urow_ref, uval_ref, 0), 0)
    lax.fori_loop(0, NREL, make_phase(vcol_ref, vrow_ref, vval_ref, NREL), 0)
    plsc.subcore_barrier()

    def wout(d, carry):
        pltpu.sync_copy(acc_ref.at[d, pl.ds(s * ROWS_PT, ROWS_PT)],
                        out_ref.at[c, d, pl.ds(s * ROWS_PT, ROWS_PT)])
        return carry
    lax.fori_loop(0, NTAB, wout, 0)


@functools.cache
def _seg_kernel_fn():
    return functools.partial(
        pl.kernel,
        mesh=plsc.VectorSubcoreMesh(core_axis_name="c", subcore_axis_name="s"),
        out_type=jax.ShapeDtypeStruct((NCORES, NTAB, N_NODE, HID), F32),
        scratch_types=[
            pltpu.VMEM_SHARED((NTAB, N_NODE, HID), F32),
            pltpu.VMEM((EPT,), jnp.int32),
            pltpu.VMEM((NCHUNK, CHUNK), jnp.int32),
            pltpu.VMEM((EPT,), F32),
            pltpu.VMEM((CHUNK, HID), F32),
            pltpu.VMEM((CHUNK, HID), F32),
            pltpu.VMEM((ROWS_PT, HID), F32),
            pltpu.SemaphoreType.DMA,
            pltpu.SemaphoreType.DMA,
            pltpu.SemaphoreType.DMA,
            pltpu.SemaphoreType.DMA,
        ],
        compiler_params=pltpu.CompilerParams(use_tc_tiling_on_sc=False),
    )(_seg_body)


def _seg_kernel(tbls, ucol, urow, uval, vcol, vrow, vval):
    return _seg_kernel_fn()(tbls, ucol, urow, uval, vcol, vrow, vval)


def _embed_body(parts_ref, sfu_ref, sfv_ref, wsu_ref, bsu_ref, wsv_ref,
                bsv_ref, wcu_ref, wcv_ref, eu_ref, ev_ref):
    def emb(off, sf_ref, ws_ref, bs_ref, wc_ref, e_ref):
        sh = jnp.maximum(
            jnp.dot(sf_ref[...], ws_ref[...], preferred_element_type=F32,
                    precision=lax.Precision.DEFAULT) + bs_ref[...], 0.0)
        acc = jnp.dot(sh, wc_ref[NREL * HID:, :],
                      preferred_element_type=F32, precision=lax.Precision.DEFAULT)
        for i in range(NREL):
            h = jnp.maximum(parts_ref[0, off + i] + parts_ref[1, off + i],
                            0.0)
            acc = acc + jnp.dot(h, wc_ref[i * HID:(i + 1) * HID, :],
                                preferred_element_type=F32, precision=lax.Precision.DEFAULT)
        e_ref[...] = jnp.maximum(acc, 0.0)

    emb(0, sfu_ref, wsu_ref, bsu_ref, wcu_ref, eu_ref)
    emb(NREL, sfv_ref, wsv_ref, bsv_ref, wcv_ref, ev_ref)


_BM = 1024


def _embed(parts, sfu, sfv, Wsu, bsu, Wsv, bsv, Wcu, Wcv):
    full = lambda *shape: pl.BlockSpec(shape, lambda m: (0,) * len(shape))
    return pl.pallas_call(
        _embed_body,
        grid=(N_NODE // _BM,),
        in_specs=[
            pl.BlockSpec((NCORES, NTAB, _BM, HID), lambda m: (0, 0, m, 0)),
            pl.BlockSpec((_BM, SHID_DIM), lambda m: (m, 0)),
            pl.BlockSpec((_BM, SHID_DIM), lambda m: (m, 0)),
            full(SHID_DIM, SHID_DIM),
            full(1, SHID_DIM),
            full(SHID_DIM, SHID_DIM),
            full(1, SHID_DIM),
            full(NREL * HID + SHID_DIM, OUT_DIM),
            full(NREL * HID + SHID_DIM, OUT_DIM),
        ],
        out_specs=[
            pl.BlockSpec((_BM, OUT_DIM), lambda m: (m, 0)),
            pl.BlockSpec((_BM, OUT_DIM), lambda m: (m, 0)),
        ],
        out_shape=[
            jax.ShapeDtypeStruct((N_NODE, OUT_DIM), F32),
            jax.ShapeDtypeStruct((N_NODE, OUT_DIM), F32),
        ],
    )(parts, sfu, sfv, Wsu, bsu, Wsv, bsv, Wcu, Wcv)


def _score_body(eu_ref, q_ref, ev_ref, o_ref, a_ref):
    @pl.when(pl.program_id(1) == 0)
    def _():
        a_ref[...] = jnp.dot(eu_ref[...], q_ref[0],
                             preferred_element_type=F32, precision=lax.Precision.DEFAULT)
    o_ref[0] = lax.dot_general(
        a_ref[...], ev_ref[...], (((1,), (1,)), ((), ())),
        preferred_element_type=F32, precision=lax.Precision.DEFAULT)


def _score(eu, ev, Q):
    return pl.pallas_call(
        _score_body,
        grid=(NREL, N_NODE // BV),
        in_specs=[
            pl.BlockSpec((N_NODE, OUT_DIM), lambda i, j: (0, 0)),
            pl.BlockSpec((1, OUT_DIM, OUT_DIM), lambda i, j: (i, 0, 0)),
            pl.BlockSpec((BV, OUT_DIM), lambda i, j: (j, 0)),
        ],
        out_specs=pl.BlockSpec((1, N_NODE, BV), lambda i, j: (i, 0, j)),
        out_shape=jax.ShapeDtypeStruct((NREL, N_NODE, N_NODE), F32),
        scratch_shapes=[pltpu.VMEM((N_NODE, OUT_DIM), F32)],
    )(eu, Q, ev)


def kernel(feature_u, feature_v, side_feature_u, side_feature_v, mu_row,
           mu_col, mu_val, mv_row, mv_col, mv_val, W, Wsu, bsu, Wsv, bsv,
           Wcu, Wcv, Q):
    tbls = _make_tables(feature_v, feature_u, W)
    r3 = (NREL, NEDGE // CHUNK, CHUNK)
    parts = _seg_kernel(tbls, mu_col, mu_row.reshape(r3), mu_val,
                        mv_col, mv_row.reshape(r3), mv_val)
    eu, ev = _embed(parts, side_feature_u, side_feature_v, Wsu,
                    bsu.reshape(1, SHID_DIM), Wsv, bsv.reshape(1, SHID_DIM),
                    Wcu, Wcv)
    return _score(eu, ev, Q)


# speedup vs baseline: 1.0816x; 1.0816x over previous
import functools

import jax
import jax.numpy as jnp
from jax import lax
from jax.experimental import pallas as pl
from jax.experimental.pallas import tpu as pltpu
from jax.experimental.pallas import tpu_sc as plsc

F32 = jnp.float32
HI = lax.Precision.HIGHEST

N_NODE = 4096
FEAT = 128
HID = 32
NREL = 5
NTAB = 2 * NREL
NEDGE = 65536
SHID_DIM = 64
OUT_DIM = 64

NCORES = 2
NSUB = 16
NTILES = NCORES * NSUB
EPT = NEDGE // NTILES
CHUNK = 128
NCHUNK = EPT // CHUNK
ROWS_PT = N_NODE // NSUB

BV = 512


def _tables_body(fv_ref, fu_ref, w_ref, o_ref):
    d = pl.program_id(0)

    @pl.when(d < NREL)
    def _():
        o_ref[0] = jnp.dot(fv_ref[...], w_ref[0], preferred_element_type=F32,
                           precision=lax.Precision.DEFAULT)

    @pl.when(d >= NREL)
    def _():
        o_ref[0] = jnp.dot(fu_ref[...], w_ref[0], preferred_element_type=F32,
                           precision=lax.Precision.DEFAULT)


def _make_tables(fv, fu, W):
    return pl.pallas_call(
        _tables_body,
        grid=(NTAB,),
        in_specs=[
            pl.BlockSpec((N_NODE, FEAT), lambda d: (0, 0)),
            pl.BlockSpec((N_NODE, FEAT), lambda d: (0, 0)),
            pl.BlockSpec((1, FEAT, HID), lambda d: (d % NREL, 0, 0)),
        ],
        out_specs=pl.BlockSpec((1, N_NODE, HID), lambda d: (d, 0, 0)),
        out_shape=jax.ShapeDtypeStruct((NTAB, N_NODE, HID), F32),
    )(fv, fu, W)


def _seg_body(tbl_ref, ucol_ref, urow_ref, uval_ref, vcol_ref, vrow_ref,
              vval_ref, out_ref,
              acc_ref, colb, rowb, valb, rbuf0, rbuf1, rbuf2, rbuf3, zbuf,
              sem0, sem1, sem2, sem3, ssem0, ssem1, ssem2, ssem3, isem):
    c = lax.axis_index("c")
    s = lax.axis_index("s")
    wid = c * NSUB + s

    def zero_row(r, carry):
        zv = jnp.zeros((16,), F32)
        zbuf[r, pl.ds(0, 16)] = zv
        zbuf[r, pl.ds(16, 16)] = zv
        return carry
    lax.fori_loop(0, ROWS_PT, zero_row, 0)

    def zero_acc(d, carry):
        pltpu.sync_copy(zbuf, acc_ref.at[d, pl.ds(s * ROWS_PT, ROWS_PT)])
        return carry
    lax.fori_loop(0, NTAB, zero_acc, 0)
    plsc.subcore_barrier()

    def scale(rbuf, k):
        for g in range(CHUNK // 16):
            v16 = valb[pl.ds(k * CHUNK + g * 16, 16)]
            for j in range(16):
                e = g * 16 + j
                sc = v16[j]
                rbuf[e, pl.ds(0, 16)] = rbuf[e, pl.ds(0, 16)] * sc
                rbuf[e, pl.ds(16, 16)] = rbuf[e, pl.ds(16, 16)] * sc

    def gather(d, k, rbuf, sem):
        pltpu.async_copy(tbl_ref.at[d].at[colb.at[pl.ds(k * CHUNK, CHUNK)]],
                         rbuf, sem)

    def gather_wait(d, k, rbuf, sem):
        pltpu.make_async_copy(
            tbl_ref.at[d].at[colb.at[pl.ds(k * CHUNK, CHUNK)]], rbuf,
            sem).wait()

    def scatter_wait(d, k, rbuf, ssem):
        pltpu.make_async_copy(rbuf, acc_ref.at[d].at[rowb.at[k]],
                              ssem).wait()

    rbufs = (rbuf0, rbuf1, rbuf2, rbuf3)
    sems = (sem0, sem1, sem2, sem3)
    ssems = (ssem0, ssem1, ssem2, ssem3)
    NB = 4

    def make_phase(col_ref, row_ref, val_ref, off):
        def phase(i, carry):
            d = i + off
            pltpu.async_copy(col_ref.at[i, pl.ds(wid * EPT, EPT)], colb,
                             isem)
            pltpu.async_copy(row_ref.at[i, pl.ds(wid * NCHUNK, NCHUNK)],
                             rowb, isem)
            pltpu.async_copy(val_ref.at[i, pl.ds(wid * EPT, EPT)], valb,
                             isem)
            pltpu.make_async_copy(col_ref.at[i, pl.ds(wid * EPT, EPT)], colb,
                                  isem).wait()
            pltpu.make_async_copy(row_ref.at[i, pl.ds(wid * NCHUNK, NCHUNK)],
                                  rowb, isem).wait()
            pltpu.make_async_copy(val_ref.at[i, pl.ds(wid * EPT, EPT)], valb,
                                  isem).wait()
            for b in range(NB):
                gather(d, b, rbufs[b], sems[b])

            def quad(q, carry2):
                for b in range(NB):
                    k = NB * q + b
                    gather_wait(d, k, rbufs[b], sems[b])
                    scale(rbufs[b], k)
                    pltpu.async_copy(rbufs[b], acc_ref.at[d].at[rowb.at[k]],
                                     ssems[b], add=True)

                    @pl.when(q < NCHUNK // NB - 1)
                    def _():
                        scatter_wait(d, k, rbufs[b], ssems[b])
                        gather(d, k + NB, rbufs[b], sems[b])
                return carry2
            lax.fori_loop(0, NCHUNK // NB, quad, 0)
            for b in range(NB):
                scatter_wait(d, NCHUNK - NB + b, rbufs[b], ssems[b])
            return carry
        return phase

    lax.fori_loop(0, NREL, make_phase(ucol_ref, urow_ref, uval_ref, 0), 0)
    lax.fori_loop(0, NREL, make_phase(vcol_ref, vrow_ref, vval_ref, NREL), 0)
    plsc.subcore_barrier()

    def wout(d, carry):
        pltpu.sync_copy(acc_ref.at[d, pl.ds(s * ROWS_PT, ROWS_PT)],
                        out_ref.at[c, d, pl.ds(s * ROWS_PT, ROWS_PT)])
        return carry
    lax.fori_loop(0, NTAB, wout, 0)


@functools.cache
def _seg_kernel_fn():
    return functools.partial(
        pl.kernel,
        mesh=plsc.VectorSubcoreMesh(core_axis_name="c", subcore_axis_name="s"),
        out_type=jax.ShapeDtypeStruct((NCORES, NTAB, N_NODE, HID), F32),
        scratch_types=[
            pltpu.VMEM_SHARED((NTAB, N_NODE, HID), F32),
            pltpu.VMEM((EPT,), jnp.int32),
            pltpu.VMEM((NCHUNK, CHUNK), jnp.int32),
            pltpu.VMEM((EPT,), F32),
            pltpu.VMEM((CHUNK, HID), F32),
            pltpu.VMEM((CHUNK, HID), F32),
            pltpu.VMEM((CHUNK, HID), F32),
            pltpu.VMEM((CHUNK, HID), F32),
            pltpu.VMEM((ROWS_PT, HID), F32),
        ] + [pltpu.SemaphoreType.DMA] * 9,
        compiler_params=pltpu.CompilerParams(use_tc_tiling_on_sc=False),
    )(_seg_body)


def _seg_kernel(tbls, ucol, urow, uval, vcol, vrow, vval):
    return _seg_kernel_fn()(tbls, ucol, urow, uval, vcol, vrow, vval)


def _embed_body(parts_ref, sfu_ref, sfv_ref, wsu_ref, bsu_ref, wsv_ref,
                bsv_ref, wcu_ref, wcv_ref, eu_ref, ev_ref):
    def emb(off, sf_ref, ws_ref, bs_ref, wc_ref, e_ref):
        sh = jnp.maximum(
            jnp.dot(sf_ref[...], ws_ref[...], preferred_element_type=F32,
                    precision=lax.Precision.DEFAULT) + bs_ref[...], 0.0)
        acc = jnp.dot(sh, wc_ref[NREL * HID:, :],
                      preferred_element_type=F32, precision=lax.Precision.DEFAULT)
        for i in range(NREL):
            h = jnp.maximum(parts_ref[0, off + i] + parts_ref[1, off + i],
                            0.0)
            acc = acc + jnp.dot(h, wc_ref[i * HID:(i + 1) * HID, :],
                                preferred_element_type=F32, precision=lax.Precision.DEFAULT)
        e_ref[...] = jnp.maximum(acc, 0.0)

    emb(0, sfu_ref, wsu_ref, bsu_ref, wcu_ref, eu_ref)
    emb(NREL, sfv_ref, wsv_ref, bsv_ref, wcv_ref, ev_ref)


_BM = 1024


def _embed(parts, sfu, sfv, Wsu, bsu, Wsv, bsv, Wcu, Wcv):
    full = lambda *shape: pl.BlockSpec(shape, lambda m: (0,) * len(shape))
    return pl.pallas_call(
        _embed_body,
        grid=(N_NODE // _BM,),
        in_specs=[
            pl.BlockSpec((NCORES, NTAB, _BM, HID), lambda m: (0, 0, m, 0)),
            pl.BlockSpec((_BM, SHID_DIM), lambda m: (m, 0)),
            pl.BlockSpec((_BM, SHID_DIM), lambda m: (m, 0)),
            full(SHID_DIM, SHID_DIM),
            full(1, SHID_DIM),
            full(SHID_DIM, SHID_DIM),
            full(1, SHID_DIM),
            full(NREL * HID + SHID_DIM, OUT_DIM),
            full(NREL * HID + SHID_DIM, OUT_DIM),
        ],
        out_specs=[
            pl.BlockSpec((_BM, OUT_DIM), lambda m: (m, 0)),
            pl.BlockSpec((_BM, OUT_DIM), lambda m: (m, 0)),
        ],
        out_shape=[
            jax.ShapeDtypeStruct((N_NODE, OUT_DIM), F32),
            jax.ShapeDtypeStruct((N_NODE, OUT_DIM), F32),
        ],
    )(parts, sfu, sfv, Wsu, bsu, Wsv, bsv, Wcu, Wcv)


def _score_body(eu_ref, q_ref, ev_ref, o_ref, a_ref):
    @pl.when(pl.program_id(1) == 0)
    def _():
        a_ref[...] = jnp.dot(eu_ref[...], q_ref[0],
                             preferred_element_type=F32, precision=lax.Precision.DEFAULT)
    o_ref[0] = lax.dot_general(
        a_ref[...], ev_ref[...], (((1,), (1,)), ((), ())),
        preferred_element_type=F32, precision=lax.Precision.DEFAULT)


def _score(eu, ev, Q):
    return pl.pallas_call(
        _score_body,
        grid=(NREL, N_NODE // BV),
        in_specs=[
            pl.BlockSpec((N_NODE, OUT_DIM), lambda i, j: (0, 0)),
            pl.BlockSpec((1, OUT_DIM, OUT_DIM), lambda i, j: (i, 0, 0)),
            pl.BlockSpec((BV, OUT_DIM), lambda i, j: (j, 0)),
        ],
        out_specs=pl.BlockSpec((1, N_NODE, BV), lambda i, j: (i, 0, j)),
        out_shape=jax.ShapeDtypeStruct((NREL, N_NODE, N_NODE), F32),
        scratch_shapes=[pltpu.VMEM((N_NODE, OUT_DIM), F32)],
    )(eu, Q, ev)


def kernel(feature_u, feature_v, side_feature_u, side_feature_v, mu_row,
           mu_col, mu_val, mv_row, mv_col, mv_val, W, Wsu, bsu, Wsv, bsv,
           Wcu, Wcv, Q):
    tbls = _make_tables(feature_v, feature_u, W)
    r3 = (NREL, NEDGE // CHUNK, CHUNK)
    parts = _seg_kernel(tbls, mu_col, mu_row.reshape(r3), mu_val,
                        mv_col, mv_row.reshape(r3), mv_val)
    eu, ev = _embed(parts, side_feature_u, side_feature_v, Wsu,
                    bsu.reshape(1, SHID_DIM), Wsv, bsv.reshape(1, SHID_DIM),
                    Wcu, Wcv)
    return _score(eu, ev, Q)
